# Initial kernel scaffold; baseline (speedup 1.0000x reference)
import jax
import jax.numpy as jnp
from jax.experimental import pallas as pl


def kernel(user, item, input_knowledge_point, theta_w, a_w, b_w):
    raise NotImplementedError("write your pallas kernel here")


# speedup vs baseline: 1.0362x; 1.0362x over previous
import dataclasses
import functools

import jax
import jax.numpy as jnp
from jax import lax
from jax.experimental import pallas as pl
from jax.experimental.pallas import tpu as pltpu
from jax.experimental.pallas import tpu_sc as plsc

K = 128
B = 16384
NC = 2
NS = 16
NW = NC * NS
ROWS_W = B // NW
CH = 128
NCH = ROWS_W // CH
GRP = 16
NGRP = CH // GRP

_mesh = plsc.VectorSubcoreMesh(core_axis_name="c", subcore_axis_name="s")

_cp = pltpu.CompilerParams()
if "needs_layout_passes" in pltpu.CompilerParams.__dataclass_fields__:
    _cp = dataclasses.replace(_cp, needs_layout_passes=False)


@functools.partial(
    pl.kernel,
    out_type=jax.ShapeDtypeStruct((B,), jnp.float32),
    mesh=_mesh,
    compiler_params=_cp,
    scratch_types=[
        pltpu.VMEM((NCH, CH), jnp.int32),
        pltpu.VMEM((NCH, CH), jnp.int32),
        pltpu.VMEM((CH, K), jnp.float32),
        pltpu.VMEM((CH, K), jnp.float32),
        pltpu.VMEM((CH, K), jnp.float32),
        pltpu.VMEM((CH,), jnp.float32),
        pltpu.VMEM((CH,), jnp.float32),
        pltpu.VMEM((GRP * GRP,), jnp.float32),
        pltpu.SemaphoreType.DMA,
    ],
)
def _irf_kernel(u_hbm, i_hbm, kp_hbm, th_hbm, a_hbm, b_hbm, out_hbm,
                uix, iix, th_v, a_v, kp_v, b_v, o_v, tr_v, sem):
    wid = lax.axis_index("c") * NS + lax.axis_index("s")
    pltpu.sync_copy(u_hbm.at[wid], uix)
    pltpu.sync_copy(i_hbm.at[wid], iix)
    lanes = lax.iota(jnp.int32, 16)

    for c in range(NCH):
        base = wid * ROWS_W + c * CH
        cp_th = pltpu.async_copy(th_hbm.at[uix.at[c]], th_v, sem)
        cp_a = pltpu.async_copy(a_hbm.at[iix.at[c]], a_v, sem)
        cp_b = pltpu.async_copy(b_hbm.at[iix.at[c]], b_v, sem)
        cp_kp = pltpu.async_copy(kp_hbm.at[pl.ds(base, CH)], kp_v, sem)
        cp_th.wait()
        cp_a.wait()
        cp_b.wait()
        cp_kp.wait()

        @pl.loop(0, NGRP)
        def _(g):
            for r in range(GRP):
                row = g * GRP + r
                acc = jnp.zeros((GRP,), jnp.float32)
                for k in range(K // GRP):
                    t = th_v[row, pl.ds(k * GRP, GRP)]
                    av = a_v[row, pl.ds(k * GRP, GRP)]
                    kv = kp_v[row, pl.ds(k * GRP, GRP)]
                    acc = acc + t * av * kv
                plsc.store_scatter(tr_v, [lanes * GRP + r], acc)
            s = tr_v[pl.ds(0, GRP)]
            for i in range(1, GRP):
                s = s + tr_v[pl.ds(i * GRP, GRP)]
            bb = b_v[pl.ds(g * GRP, GRP)]
            o_v[pl.ds(g * GRP, GRP)] = 1.0 / (1.0 + jnp.exp(bb - s))

        pltpu.sync_copy(o_v, out_hbm.at[pl.ds(base, CH)])


@jax.jit
def kernel(user, item, input_knowledge_point, theta_w, a_w, b_w):
    u2 = user.astype(jnp.int32).reshape(NW, NCH, CH)
    i2 = item.astype(jnp.int32).reshape(NW, NCH, CH)
    b_flat = b_w.reshape(-1)
    return _irf_kernel(u2, i2, input_knowledge_point, theta_w, a_w, b_flat)


# speedup vs baseline: 1.1940x; 1.1523x over previous
import dataclasses
import functools

import jax
import jax.numpy as jnp
from jax import lax
from jax.experimental import pallas as pl
from jax.experimental.pallas import tpu as pltpu
from jax.experimental.pallas import tpu_sc as plsc

K = 128
B = 16384
NC = 2
NS = 16
NW = NC * NS
ROWS_W = B // NW
CH = 128
NCH = ROWS_W // CH
GRP = 16
NGRP = CH // GRP

_mesh = plsc.VectorSubcoreMesh(core_axis_name="c", subcore_axis_name="s")

_cp = pltpu.CompilerParams()
if "needs_layout_passes" in pltpu.CompilerParams.__dataclass_fields__:
    _cp = dataclasses.replace(_cp, needs_layout_passes=False)


@functools.partial(
    pl.kernel,
    out_type=jax.ShapeDtypeStruct((B,), jnp.float32),
    mesh=_mesh,
    compiler_params=_cp,
    scratch_types=[
        pltpu.VMEM((NCH, CH), jnp.int32),
        pltpu.VMEM((NCH, CH), jnp.int32),
        pltpu.VMEM((2, CH, K), jnp.float32),
        pltpu.VMEM((2, CH, K), jnp.float32),
        pltpu.VMEM((2, CH, K), jnp.float32),
        pltpu.VMEM((2, CH), jnp.float32),
        pltpu.VMEM((2, CH), jnp.float32),
        pltpu.VMEM((GRP * GRP,), jnp.float32),
        pltpu.SemaphoreType.DMA,
        pltpu.SemaphoreType.DMA,
        pltpu.SemaphoreType.DMA,
        pltpu.SemaphoreType.DMA,
    ],
)
def _irf_kernel(u_hbm, i_hbm, kp_hbm, th_hbm, a_hbm, b_hbm, out_hbm,
                uix, iix, th_v, a_v, kp_v, b_v, o_v, tr_v,
                sem0, sem1, osem0, osem1):
    wid = lax.axis_index("c") * NS + lax.axis_index("s")
    pltpu.sync_copy(u_hbm.at[wid], uix)
    pltpu.sync_copy(i_hbm.at[wid], iix)
    lanes = lax.iota(jnp.int32, 16)
    sems = (sem0, sem1)
    osems = (osem0, osem1)

    def issue(c):
        p = c % 2
        base = wid * ROWS_W + c * CH
        return (
            pltpu.async_copy(th_hbm.at[uix.at[c]], th_v.at[p], sems[p]),
            pltpu.async_copy(a_hbm.at[iix.at[c]], a_v.at[p], sems[p]),
            pltpu.async_copy(b_hbm.at[iix.at[c]], b_v.at[p], sems[p]),
            pltpu.async_copy(kp_hbm.at[pl.ds(base, CH)], kp_v.at[p], sems[p]),
        )

    pending = issue(0)
    out_pending = [None, None]
    for c in range(NCH):
        p = c % 2
        nxt = issue(c + 1) if c + 1 < NCH else None
        for cp in pending:
            cp.wait()
        pending = nxt
        th_c, a_c, kp_c, b_c, o_c = (
            th_v.at[p], a_v.at[p], kp_v.at[p], b_v.at[p], o_v.at[p])
        if out_pending[p] is not None:
            out_pending[p].wait()
            out_pending[p] = None

        @pl.loop(0, NGRP)
        def _(g):
            for r in range(GRP):
                row = g * GRP + r
                acc = jnp.zeros((GRP,), jnp.float32)
                for k in range(K // GRP):
                    t = th_c[row, pl.ds(k * GRP, GRP)]
                    av = a_c[row, pl.ds(k * GRP, GRP)]
                    kv = kp_c[row, pl.ds(k * GRP, GRP)]
                    acc = acc + t * av * kv
                plsc.store_scatter(tr_v, [lanes * GRP + r], acc)
            s = tr_v[pl.ds(0, GRP)]
            for i in range(1, GRP):
                s = s + tr_v[pl.ds(i * GRP, GRP)]
            bb = b_c[pl.ds(g * GRP, GRP)]
            o_c[pl.ds(g * GRP, GRP)] = 1.0 / (1.0 + jnp.exp(bb - s))

        base = wid * ROWS_W + c * CH
        out_pending[p] = pltpu.async_copy(
            o_c, out_hbm.at[pl.ds(base, CH)], osems[p])
    for cp in out_pending:
        if cp is not None:
            cp.wait()


@jax.jit
def kernel(user, item, input_knowledge_point, theta_w, a_w, b_w):
    u2 = user.astype(jnp.int32).reshape(NW, NCH, CH)
    i2 = item.astype(jnp.int32).reshape(NW, NCH, CH)
    b_flat = b_w.reshape(-1)
    return _irf_kernel(u2, i2, input_knowledge_point, theta_w, a_w, b_flat)


# speedup vs baseline: 1.3021x; 1.0906x over previous
import dataclasses
import functools

import jax
import jax.numpy as jnp
from jax import lax
from jax.experimental import pallas as pl
from jax.experimental.pallas import tpu as pltpu
from jax.experimental.pallas import tpu_sc as plsc

K = 128
B = 16384
NC = 2
NS = 16
NW = NC * NS
ROWS_W = B // NW
CH = 128
NCH = ROWS_W // CH
GRP = 16
NGRP = CH // GRP

_mesh = plsc.VectorSubcoreMesh(core_axis_name="c", subcore_axis_name="s")

_cp = pltpu.CompilerParams()
if "needs_layout_passes" in pltpu.CompilerParams.__dataclass_fields__:
    _cp = dataclasses.replace(_cp, needs_layout_passes=False)


@functools.partial(
    pl.kernel,
    out_type=jax.ShapeDtypeStruct((B,), jnp.float32),
    mesh=_mesh,
    compiler_params=_cp,
    scratch_types=[
        pltpu.VMEM((ROWS_W,), jnp.int32),
        pltpu.VMEM((ROWS_W,), jnp.int32),
        pltpu.VMEM((2, CH, K), jnp.float32),
        pltpu.VMEM((2, CH, K), jnp.float32),
        pltpu.VMEM((2, CH, K), jnp.float32),
        pltpu.VMEM((2, CH), jnp.float32),
        pltpu.VMEM((2, CH), jnp.float32),
        pltpu.VMEM((GRP * GRP,), jnp.float32),
        pltpu.SemaphoreType.DMA((2,)),
        pltpu.SemaphoreType.DMA((2,)),
    ],
)
def _irf_kernel(u_hbm, i_hbm, kp_hbm, th_hbm, a_hbm, b_hbm, out_hbm,
                uix, iix, th_v, a_v, kp_v, b_v, o_v, tr_v, isem, osem):
    wid = lax.axis_index("c") * NS + lax.axis_index("s")
    base_w = wid * ROWS_W
    pltpu.sync_copy(u_hbm.at[pl.ds(base_w, ROWS_W)], uix)
    pltpu.sync_copy(i_hbm.at[pl.ds(base_w, ROWS_W)], iix)
    lanes = lax.iota(jnp.int32, 16)

    def issue(c, p):
        pltpu.async_copy(th_hbm.at[uix.at[pl.ds(c * CH, CH)]],
                         th_v.at[p], isem.at[p])
        pltpu.async_copy(a_hbm.at[iix.at[pl.ds(c * CH, CH)]],
                         a_v.at[p], isem.at[p])
        pltpu.async_copy(b_hbm.at[iix.at[pl.ds(c * CH, CH)]],
                         b_v.at[p], isem.at[p])
        pltpu.async_copy(kp_hbm.at[pl.ds(base_w + c * CH, CH)],
                         kp_v.at[p], isem.at[p])

    def wait_gathers(p):
        pltpu.make_async_copy(th_hbm.at[pl.ds(0, CH)], th_v.at[p],
                              isem.at[p]).wait()
        pltpu.make_async_copy(a_hbm.at[pl.ds(0, CH)], a_v.at[p],
                              isem.at[p]).wait()
        pltpu.make_async_copy(b_hbm.at[pl.ds(0, CH)], b_v.at[p],
                              isem.at[p]).wait()
        pltpu.make_async_copy(kp_hbm.at[pl.ds(0, CH)], kp_v.at[p],
                              isem.at[p]).wait()

    issue(0, 0)

    @pl.loop(0, NCH)
    def _(c):
        p = lax.rem(c, 2)

        @pl.when(c + 1 < NCH)
        def _():
            issue(c + 1, 1 - p)

        wait_gathers(p)

        @pl.when(c >= 2)
        def _():
            pltpu.make_async_copy(o_v.at[p], out_hbm.at[pl.ds(0, CH)],
                                  osem.at[p]).wait()

        @pl.loop(0, NGRP)
        def _(g):
            for r in range(GRP):
                row = g * GRP + r
                acc = jnp.zeros((GRP,), jnp.float32)
                for k in range(K // GRP):
                    t = th_v[p, row, pl.ds(k * GRP, GRP)]
                    av = a_v[p, row, pl.ds(k * GRP, GRP)]
                    kv = kp_v[p, row, pl.ds(k * GRP, GRP)]
                    acc = acc + t * av * kv
                plsc.store_scatter(tr_v, [lanes * GRP + r], acc)
            s = tr_v[pl.ds(0, GRP)]
            for i in range(1, GRP):
                s = s + tr_v[pl.ds(i * GRP, GRP)]
            bb = b_v[p, pl.ds(g * GRP, GRP)]
            o_v[p, pl.ds(g * GRP, GRP)] = 1.0 / (1.0 + jnp.exp(bb - s))

        pltpu.async_copy(o_v.at[p], out_hbm.at[pl.ds(base_w + c * CH, CH)],
                         osem.at[p])

    for p in range(2):
        pltpu.make_async_copy(o_v.at[p], out_hbm.at[pl.ds(0, CH)],
                              osem.at[p]).wait()


@jax.jit
def kernel(user, item, input_knowledge_point, theta_w, a_w, b_w):
    return _irf_kernel(user.astype(jnp.int32), item.astype(jnp.int32),
                       input_knowledge_point, theta_w, a_w, b_w.reshape(-1))
